# baseline (speedup 1.0000x reference)
import functools

import jax
import jax.numpy as jnp
from jax import lax
from jax.experimental import pallas as pl
from jax.experimental.pallas import tpu as pltpu
from jax.experimental.pallas import tpu_sc as plsc

H = 2048
W = 2048
C = 8
NW = 32
ROWS_PER_TILE = H // NW
CHUNK_ROWS = 4
IN_W = CHUNK_ROWS * W * C
OUT_W = CHUNK_ROWS * W
NCHUNK = ROWS_PER_TILE // CHUNK_ROWS
TILE_IN = ROWS_PER_TILE * W * C
TILE_OUT = ROWS_PER_TILE * W


def _body(in_hbm, cost_hbm, weight_hbm, out_hbm, cost_out_hbm,
          inbuf, outbuf, wbuf, cbuf, cobuf):
    wid = lax.axis_index("s") * 2 + lax.axis_index("c")

    pltpu.sync_copy(weight_hbm, wbuf)
    pltpu.sync_copy(cost_hbm, cbuf)
    w = wbuf[...]

    ws = [w[i] for i in range(C)]
    mx = functools.reduce(jnp.maximum, ws)
    e = jnp.exp(w - mx)
    es = [e[i] for i in range(C)]
    s = functools.reduce(lambda a, b: a + b, es)
    p = e / s
    ps = [p[i] for i in range(C)]

    best = ps[0]
    sel = jnp.int32(0)
    for i in range(1, C):
        gt = ps[i] > best
        best = jnp.where(gt, ps[i], best)
        sel = jnp.where(gt, jnp.int32(i), sel)

    iota = lax.broadcasted_iota(jnp.int32, (16,), 0)
    pat = iota * C + sel

    @pl.when(wid == 0)
    def _():
        cv = cbuf[...]
        co = cv[0]
        for i in range(1, C):
            co = jnp.where(sel == i, cv[i], co)
        cobuf[...] = jnp.full((16,), co, jnp.float32)
        pltpu.sync_copy(cobuf, cost_out_hbm)

    tile_in = wid * TILE_IN
    tile_out = wid * TILE_OUT

    def chunk_body(ci, carry):
        in_off = tile_in + ci * IN_W
        out_off = tile_out + ci * OUT_W
        pltpu.sync_copy(in_hbm.at[pl.ds(in_off, IN_W)], inbuf)

        def jbody(j, c2):
            g = plsc.load_gather(inbuf, [pat + j * 128])
            outbuf[pl.ds(j * 16, 16)] = g
            return c2

        lax.fori_loop(0, OUT_W // 16, jbody, 0)
        pltpu.sync_copy(outbuf, out_hbm.at[pl.ds(out_off, OUT_W)])
        return carry

    lax.fori_loop(0, NCHUNK, chunk_body, 0)


def kernel(input, cost, weight):
    in_flat = input.reshape(H * W * C)
    w16 = jnp.pad(weight, (0, 8), constant_values=-jnp.inf)
    c16 = jnp.pad(cost, (0, 8))

    mesh = plsc.VectorSubcoreMesh(core_axis_name="c", subcore_axis_name="s")
    out_flat, cost_out = pl.kernel(
        _body,
        out_type=[
            jax.ShapeDtypeStruct((H * W,), jnp.float32),
            jax.ShapeDtypeStruct((16,), jnp.float32),
        ],
        mesh=mesh,
        compiler_params=pltpu.CompilerParams(needs_layout_passes=False),
        scratch_types=[
            pltpu.VMEM((IN_W,), jnp.float32),
            pltpu.VMEM((OUT_W,), jnp.float32),
            pltpu.VMEM((16,), jnp.float32),
            pltpu.VMEM((16,), jnp.float32),
            pltpu.VMEM((16,), jnp.float32),
        ],
    )(in_flat, c16, w16)

    return out_flat.reshape(H, W), cost_out[0]


# speedup vs baseline: 43.5897x; 43.5897x over previous
import functools

import jax
import jax.numpy as jnp
from jax import lax
from jax.experimental import pallas as pl
from jax.experimental.pallas import tpu as pltpu
from jax.experimental.pallas import tpu_sc as plsc

H = 2048
W = 2048
C = 8
NW = 32
ROWS_PER_TILE = H // NW
BANDS_PER_TILE = ROWS_PER_TILE // 8
NSLOT = 4


def _body(in_hbm, cost_hbm, weight_hbm, out_hbm, cost_out_hbm,
          bandbuf, wbuf, cbuf, cobuf, *sems):
    in_sems = sems[:NSLOT]
    out_sems = sems[NSLOT:]
    wid = lax.axis_index("s") * 2 + lax.axis_index("c")

    pltpu.sync_copy(weight_hbm, wbuf)
    pltpu.sync_copy(cost_hbm, cbuf)
    w = wbuf[...]

    ws = [w[i] for i in range(C)]
    mx = functools.reduce(jnp.maximum, ws)
    e = jnp.exp(w - mx)
    es = [e[i] for i in range(C)]
    s = functools.reduce(lambda a, b: a + b, es)
    p = e / s
    ps = [p[i] for i in range(C)]

    best = ps[0]
    sel = jnp.int32(0)
    for i in range(1, C):
        gt = ps[i] > best
        best = jnp.where(gt, ps[i], best)
        sel = jnp.where(gt, jnp.int32(i), sel)

    @pl.when(wid == 0)
    def _():
        cv = cbuf[...]
        co = cv[0]
        for i in range(1, C):
            co = jnp.where(sel == i, cv[i], co)
        cobuf[...] = jnp.full((16,), co, jnp.float32)
        pltpu.sync_copy(cobuf, cost_out_hbm)

    row0 = wid * ROWS_PER_TILE
    band0 = wid * BANDS_PER_TILE

    def start_ins(band, slot):
        handles = []
        for r in range(8):
            row = row0 + band * 8 + r
            h = pltpu.async_copy(
                in_hbm.at[row, :, pl.ds(sel, 1), :],
                bandbuf.at[slot, :, pl.ds(r, 1), :],
                in_sems[slot],
            )
            handles.append(h)
        return handles

    in_handles = {0: start_ins(0, 0)}
    out_handles = {}

    for band in range(BANDS_PER_TILE):
        slot = band % NSLOT
        nxt = band + 1
        if nxt < BANDS_PER_TILE:
            nslot = nxt % NSLOT
            if nxt >= NSLOT:
                out_handles[nxt - NSLOT].wait()
            in_handles[nxt] = start_ins(nxt, nslot)
        for h in in_handles[band]:
            h.wait()
        out_handles[band] = pltpu.async_copy(
            bandbuf.at[slot], out_hbm.at[band0 + band], out_sems[slot]
        )

    for band in range(BANDS_PER_TILE - NSLOT, BANDS_PER_TILE):
        out_handles[band].wait()


def kernel(input, cost, weight):
    in4 = input.reshape(H, 16, 128, C).transpose(0, 1, 3, 2)
    w16 = jnp.pad(weight, (0, 8), constant_values=-jnp.inf)
    c16 = jnp.pad(cost, (0, 8))

    mesh = plsc.VectorSubcoreMesh(core_axis_name="c", subcore_axis_name="s")
    out4, cost_out = pl.kernel(
        _body,
        out_type=[
            jax.ShapeDtypeStruct((H // 8, 16, 8, 128), jnp.float32),
            jax.ShapeDtypeStruct((16,), jnp.float32),
        ],
        mesh=mesh,
        compiler_params=pltpu.CompilerParams(needs_layout_passes=False),
        scratch_types=(
            [
                pltpu.VMEM((NSLOT, 16, 8, 128), jnp.float32),
                pltpu.VMEM((16,), jnp.float32),
                pltpu.VMEM((16,), jnp.float32),
                pltpu.VMEM((16,), jnp.float32),
            ]
            + [pltpu.SemaphoreType.DMA] * (2 * NSLOT)
        ),
    )(in4, c16, w16)

    output = out4.transpose(0, 2, 1, 3).reshape(H, W)
    return output, cost_out[0]


# speedup vs baseline: 47.7065x; 1.0944x over previous
import functools

import jax
import jax.numpy as jnp
from jax import lax
from jax.experimental import pallas as pl
from jax.experimental.pallas import tpu as pltpu
from jax.experimental.pallas import tpu_sc as plsc

H = 2048
W = 2048
C = 8
NW = 32
ROWS_PER_TILE = H // NW
BANDS_PER_TILE = ROWS_PER_TILE // 8
NSLOT = 3


def _body(in_hbm, cost_hbm, weight_hbm, out_hbm, cost_out_hbm,
          bandbuf, spbuf, wbuf, cbuf, cobuf, *sems):
    in_sems = sems[:NSLOT]
    out_sems = sems[NSLOT:2 * NSLOT]
    s_in_sems = sems[2 * NSLOT:3 * NSLOT]
    s_out_sems = sems[3 * NSLOT:]
    wid = lax.axis_index("s") * 2 + lax.axis_index("c")

    pltpu.sync_copy(weight_hbm, wbuf.at[pl.ds(0, C)])
    pltpu.sync_copy(cost_hbm, cbuf.at[pl.ds(0, C)])
    w = wbuf[...]

    ws = [w[i] for i in range(C)]
    mx = functools.reduce(jnp.maximum, ws)
    e = jnp.exp(w - mx)
    es = [e[i] for i in range(C)]
    s = functools.reduce(lambda a, b: a + b, es)
    p = e / s
    ps = [p[i] for i in range(C)]

    best = ps[0]
    sel = jnp.int32(0)
    for i in range(1, C):
        gt = ps[i] > best
        best = jnp.where(gt, ps[i], best)
        sel = jnp.where(gt, jnp.int32(i), sel)

    @pl.when(wid == 0)
    def _():
        cv = cbuf[...]
        co = cv[0]
        for i in range(1, C):
            co = jnp.where(sel == i, cv[i], co)
        cobuf[...] = jnp.full((16,), co, jnp.float32)
        pltpu.sync_copy(cobuf, cost_out_hbm)

    row0 = wid * ROWS_PER_TILE
    band0 = wid * BANDS_PER_TILE
    sid = lax.axis_index("s")

    def start_ins(band, dst, sem):
        handles = []
        for r in range(8):
            row = row0 + band * 8 + r
            h = pltpu.async_copy(
                in_hbm.at[row, :, pl.ds(sel, 1), :],
                dst.at[:, pl.ds(r, 1), :],
                sem,
            )
            handles.append(h)
        return handles

    half = BANDS_PER_TILE // 2

    def t_buf(k):
        return bandbuf.at[k % NSLOT]

    def s_buf(k):
        return spbuf.at[sid, k % NSLOT]

    ins_t, ins_s, outs_t, outs_s = {}, {}, {}, {}

    def start_t(k):
        ins_t[k] = start_ins(2 * k, t_buf(k), in_sems[k % NSLOT])

    def start_s(k):
        ins_s[k] = start_ins(2 * k + 1, s_buf(k), s_in_sems[k % NSLOT])

    for k in range(min(NSLOT, half)):
        start_t(k)
        start_s(k)

    for k in range(half):
        for h in ins_t[k]:
            h.wait()
        outs_t[k] = pltpu.async_copy(
            t_buf(k), out_hbm.at[band0 + 2 * k], out_sems[k % NSLOT]
        )
        for h in ins_s[k]:
            h.wait()
        outs_s[k] = pltpu.async_copy(
            s_buf(k), out_hbm.at[band0 + 2 * k + 1], s_out_sems[k % NSLOT]
        )
        nxt = k + NSLOT
        if nxt < half:
            outs_t[k].wait()
            start_t(nxt)
            outs_s[k].wait()
            start_s(nxt)

    for k in range(max(0, half - NSLOT), half):
        outs_t[k].wait()
        outs_s[k].wait()


def kernel(input, cost, weight):
    in4 = input.reshape(H, 16, 128, C).transpose(0, 1, 3, 2)

    mesh = plsc.VectorSubcoreMesh(core_axis_name="c", subcore_axis_name="s")
    out4, cost_out = pl.kernel(
        _body,
        out_type=[
            jax.ShapeDtypeStruct((H // 8, 16, 8, 128), jnp.float32),
            jax.ShapeDtypeStruct((16,), jnp.float32),
        ],
        mesh=mesh,
        compiler_params=pltpu.CompilerParams(needs_layout_passes=False),
        scratch_types=(
            [
                pltpu.VMEM((NSLOT, 16, 8, 128), jnp.float32),
                pltpu.VMEM_SHARED((16, NSLOT, 16, 8, 128), jnp.float32),
                pltpu.VMEM((16,), jnp.float32),
                pltpu.VMEM((16,), jnp.float32),
                pltpu.VMEM((16,), jnp.float32),
            ]
            + [pltpu.SemaphoreType.DMA] * (4 * NSLOT)
        ),
    )(in4, cost, weight)

    output = out4.transpose(0, 2, 1, 3).reshape(H, W)
    return output, cost_out[0]


# speedup vs baseline: 47.7433x; 1.0008x over previous
import functools

import jax
import jax.numpy as jnp
from jax import lax
from jax.experimental import pallas as pl
from jax.experimental.pallas import tpu as pltpu
from jax.experimental.pallas import tpu_sc as plsc

H = 2048
W = 2048
C = 8
NW = 32
ROWS_PER_TILE = H // NW
BANDS_PER_TILE = ROWS_PER_TILE // 8
NSLOT = 3


def _body(in_hbm, cost_hbm, weight_hbm, out_hbm, cost_out_hbm,
          bandbuf, spbuf, wbuf, cbuf, cobuf, *sems):
    in_sems = sems[:NSLOT]
    out_sems = sems[NSLOT:2 * NSLOT]
    s_in_sems = sems[2 * NSLOT:3 * NSLOT]
    s_out_sems = sems[3 * NSLOT:]
    wid = lax.axis_index("s") * 2 + lax.axis_index("c")

    pltpu.sync_copy(weight_hbm, wbuf.at[pl.ds(0, C)])
    pltpu.sync_copy(cost_hbm, cbuf.at[pl.ds(0, C)])
    w = wbuf[...]

    ws = [w[i] for i in range(C)]
    mx = functools.reduce(jnp.maximum, ws)
    e = jnp.exp(w - mx)
    es = [e[i] for i in range(C)]
    s = functools.reduce(lambda a, b: a + b, es)
    p = e / s
    ps = [p[i] for i in range(C)]

    best = ps[0]
    sel = jnp.int32(0)
    for i in range(1, C):
        gt = ps[i] > best
        best = jnp.where(gt, ps[i], best)
        sel = jnp.where(gt, jnp.int32(i), sel)

    @pl.when(wid == 0)
    def _():
        cv = cbuf[...]
        co = cv[0]
        for i in range(1, C):
            co = jnp.where(sel == i, cv[i], co)
        cobuf[...] = jnp.full((16,), co, jnp.float32)
        pltpu.sync_copy(cobuf, cost_out_hbm)

    row0 = wid * ROWS_PER_TILE
    band0 = wid * BANDS_PER_TILE
    sid = lax.axis_index("s")

    def start_ins(band, dst, sem):
        handles = []
        for r in range(8):
            row = row0 + band * 8 + r
            h = pltpu.async_copy(
                in_hbm.at[row, :, pl.ds(sel, 1), :],
                dst.at[:, pl.ds(r, 1), :],
                sem,
            )
            handles.append(h)
        return handles

    half = BANDS_PER_TILE // 2

    def t_buf(k):
        return bandbuf.at[k % NSLOT]

    def s_buf(k):
        return spbuf.at[sid, k % NSLOT]

    ins_t, ins_s, outs_t, outs_s = {}, {}, {}, {}

    def start_t(k):
        ins_t[k] = start_ins(2 * k, t_buf(k), in_sems[k % NSLOT])

    def start_s(k):
        ins_s[k] = start_ins(2 * k + 1, s_buf(k), s_in_sems[k % NSLOT])

    for k in range(min(NSLOT, half)):
        start_t(k)
        start_s(k)

    for k in range(half):
        for h in ins_t[k]:
            h.wait()
        outs_t[k] = pltpu.async_copy(
            t_buf(k), out_hbm.at[band0 + 2 * k], out_sems[k % NSLOT]
        )
        for h in ins_s[k]:
            h.wait()
        outs_s[k] = pltpu.async_copy(
            s_buf(k), out_hbm.at[band0 + 2 * k + 1], s_out_sems[k % NSLOT]
        )
        nxt = k + NSLOT
        if nxt < half:
            outs_t[k].wait()
            start_t(nxt)
            outs_s[k].wait()
            start_s(nxt)

    for k in range(max(0, half - NSLOT), half):
        outs_t[k].wait()
        outs_s[k].wait()


def kernel(input, cost, weight):
    in4 = input.reshape(H, 16, 128, C).transpose(0, 1, 3, 2)

    mesh = plsc.VectorSubcoreMesh(core_axis_name="c", subcore_axis_name="s")
    out4, cost_out = pl.kernel(
        _body,
        out_type=[
            jax.ShapeDtypeStruct((H // 8, 16, 8, 128), jnp.float32),
            jax.ShapeDtypeStruct((16,), jnp.float32),
        ],
        mesh=mesh,
        compiler_params=pltpu.CompilerParams(
            needs_layout_passes=False,
            disable_bounds_checks=True,
            skip_device_barrier=True,
        ),
        scratch_types=(
            [
                pltpu.VMEM((NSLOT, 16, 8, 128), jnp.float32),
                pltpu.VMEM_SHARED((16, NSLOT, 16, 8, 128), jnp.float32),
                pltpu.VMEM((16,), jnp.float32),
                pltpu.VMEM((16,), jnp.float32),
                pltpu.VMEM((16,), jnp.float32),
            ]
            + [pltpu.SemaphoreType.DMA] * (4 * NSLOT)
        ),
    )(in4, cost, weight)

    output = out4.transpose(0, 2, 1, 3).reshape(H, W)
    return output, cost_out[0]
